# Initial kernel scaffold; baseline (speedup 1.0000x reference)
import jax
import jax.numpy as jnp
from jax.experimental import pallas as pl


def kernel(x, edge_index, W, att_src, att_dst, bias, bn_gamma, bn_beta, ens_W, ens_b):
    raise NotImplementedError("write your pallas kernel here")


# speedup vs baseline: 22.4187x; 22.4187x over previous
import functools

import jax
import jax.numpy as jnp
from jax import lax
from jax.experimental import pallas as pl
from jax.experimental.pallas import tpu as pltpu
from jax.experimental.pallas import tpu_sc as plsc

N = 10000
E = 320000
D = 128
H = 8
C = 2
BN_EPS = 1e-5

EP = E + N
NC, NS, LANES = 2, 16, 16
CHUNK = 128
NCHUNK = -(-EP // (NS * CHUNK))
EP_PAD = NCHUNK * NS * CHUNK
RC = EP_PAD // CHUNK
HPC = H // NC
ROWS_T = N // NS
DW = D + 16

BN = 2000
NB = N // BN


def _alpha_accum(ref, vals, hd):
    col = lax.broadcasted_iota(jnp.int32, (BN, H), 1) == hd

    @pl.when(hd == 0)
    def _():
        ref[...] = jnp.zeros((BN, H), jnp.float32)

    ref[...] += jnp.where(col, vals[:, None], 0.0)


def _pre1_body(x_ref, w_ref, avs_ref, avd_ref, xw_ref, asq_ref, adq_ref):
    hd = pl.program_id(1)
    xw = jnp.dot(x_ref[...], w_ref[0], preferred_element_type=jnp.float32)
    xw_ref[0] = xw
    _alpha_accum(asq_ref, jnp.sum(xw * avs_ref[0, 0][None, :], axis=1), hd)
    _alpha_accum(adq_ref, jnp.sum(xw * avd_ref[0, 0][None, :], axis=1), hd)


def _pre2_body(feat_ref, den_ref, w_ref, avs_ref, avd_ref, sc_ref, sh_ref,
               xw_ref, asq_ref, adq_ref):
    hd = pl.program_id(1)
    feat = feat_ref[0]
    den = den_ref[0][:, 0:1]
    h1 = feat / den
    h1 = h1 * sc_ref[0, 0][None, :] + sh_ref[0, 0][None, :]
    h1 = jnp.where(h1 > 0, h1, jnp.exp(h1) - 1.0)
    xw = jnp.dot(h1, w_ref[0], preferred_element_type=jnp.float32)
    xw_ref[0] = xw
    _alpha_accum(asq_ref, jnp.sum(xw * avs_ref[0, 0][None, :], axis=1), hd)
    _alpha_accum(adq_ref, jnp.sum(xw * avd_ref[0, 0][None, :], axis=1), hd)


def _final_body(feat_ref, den_ref, sc_ref, sh_ref, ew_ref, eb_ref, out_ref):
    hd = pl.program_id(1)
    feat = feat_ref[0]
    den = den_ref[0][:, 0:1]
    h2 = feat / den
    h2 = h2 * sc_ref[0, 0][None, :] + sh_ref[0, 0][None, :]
    h2 = jnp.where(h2 > 0, h2, jnp.exp(h2) - 1.0)
    part = jnp.dot(h2, ew_ref[0], preferred_element_type=jnp.float32)

    @pl.when(hd == 0)
    def _():
        out_ref[...] = jnp.broadcast_to(eb_ref[...][None, :], (BN, C))

    out_ref[...] += part


def _tc_pre1(x, Wl, avs, avd):
    return pl.pallas_call(
        _pre1_body,
        grid=(NB, H),
        in_specs=[
            pl.BlockSpec((BN, D), lambda i, h: (i, 0)),
            pl.BlockSpec((1, D, D), lambda i, h: (h, 0, 0)),
            pl.BlockSpec((1, 1, D), lambda i, h: (h, 0, 0)),
            pl.BlockSpec((1, 1, D), lambda i, h: (h, 0, 0)),
        ],
        out_specs=[
            pl.BlockSpec((1, BN, D), lambda i, h: (h, i, 0)),
            pl.BlockSpec((BN, H), lambda i, h: (i, 0)),
            pl.BlockSpec((BN, H), lambda i, h: (i, 0)),
        ],
        out_shape=[
            jax.ShapeDtypeStruct((H, N, D), jnp.float32),
            jax.ShapeDtypeStruct((N, H), jnp.float32),
            jax.ShapeDtypeStruct((N, H), jnp.float32),
        ],
    )(x, Wl, avs.reshape(H, 1, D), avd.reshape(H, 1, D))


def _tc_pre2(feat, den, Wl, avs, avd, scale, shift):
    return pl.pallas_call(
        _pre2_body,
        grid=(NB, H),
        in_specs=[
            pl.BlockSpec((1, BN, D), lambda i, h: (h, i, 0)),
            pl.BlockSpec((1, BN, LANES), lambda i, h: (h, i, 0)),
            pl.BlockSpec((1, D, D), lambda i, h: (h, 0, 0)),
            pl.BlockSpec((1, 1, D), lambda i, h: (h, 0, 0)),
            pl.BlockSpec((1, 1, D), lambda i, h: (h, 0, 0)),
            pl.BlockSpec((1, 1, D), lambda i, h: (h, 0, 0)),
            pl.BlockSpec((1, 1, D), lambda i, h: (h, 0, 0)),
        ],
        out_specs=[
            pl.BlockSpec((1, BN, D), lambda i, h: (h, i, 0)),
            pl.BlockSpec((BN, H), lambda i, h: (i, 0)),
            pl.BlockSpec((BN, H), lambda i, h: (i, 0)),
        ],
        out_shape=[
            jax.ShapeDtypeStruct((H, N, D), jnp.float32),
            jax.ShapeDtypeStruct((N, H), jnp.float32),
            jax.ShapeDtypeStruct((N, H), jnp.float32),
        ],
    )(feat, den, Wl, avs.reshape(H, 1, D), avd.reshape(H, 1, D), scale.reshape(H, 1, D), shift.reshape(H, 1, D))


def _tc_final(feat, den, scale, shift, ensW, ensb):
    return pl.pallas_call(
        _final_body,
        grid=(NB, H),
        in_specs=[
            pl.BlockSpec((1, BN, D), lambda i, h: (h, i, 0)),
            pl.BlockSpec((1, BN, LANES), lambda i, h: (h, i, 0)),
            pl.BlockSpec((1, 1, D), lambda i, h: (h, 0, 0)),
            pl.BlockSpec((1, 1, D), lambda i, h: (h, 0, 0)),
            pl.BlockSpec((1, D, C), lambda i, h: (h, 0, 0)),
            pl.BlockSpec((C,), lambda i, h: (0,)),
        ],
        out_specs=pl.BlockSpec((BN, C), lambda i, h: (i, 0)),
        out_shape=jax.ShapeDtypeStruct((N, C), jnp.float32),
    )(feat, den, scale.reshape(H, 1, D), shift.reshape(H, 1, D), ensW, ensb)


def _sc_edge_body(xw2d, asl, adl, src2d, dst2d, feat, den,
                  acc_sh, den_sh, as_t, ad_t, srcbuf, dstbuf, rows_st,
                  eerow, eebuf, sem):
    core = lax.axis_index("c")
    sid = lax.axis_index("s")

    for hl in range(HPC):
        head = core * HPC + hl

        def zrow(r, _):
            for k in range(D // LANES):
                rows_st[r, pl.ds(k * LANES, LANES)] = jnp.zeros(
                    (LANES,), jnp.float32)
            eerow[r, pl.ds(0, LANES)] = jnp.zeros((LANES,), jnp.float32)
            return 0
        lax.fori_loop(0, CHUNK, zrow, 0)
        for k in range(5):
            pltpu.sync_copy(rows_st.at[pl.ds(0, 125)],
                            acc_sh.at[pl.ds(sid * ROWS_T + k * 125, 125)])
            pltpu.sync_copy(eerow.at[pl.ds(0, 125)],
                            den_sh.at[pl.ds(sid * ROWS_T + k * 125, 125)])

        pltpu.sync_copy(asl.at[head], as_t)
        pltpu.sync_copy(adl.at[head], ad_t)

        plsc.subcore_barrier()

        def chunk_body(j, _):
            row = sid * NCHUNK + j
            pltpu.sync_copy(src2d.at[row], srcbuf)
            pltpu.sync_copy(dst2d.at[row], dstbuf.at[0])

            base = row * CHUNK
            for k in range(CHUNK // LANES):
                sv = srcbuf[pl.ds(k * LANES, LANES)]
                dv = dstbuf[0, pl.ds(k * LANES, LANES)]
                av = plsc.load_gather(as_t, [sv])
                bv = plsc.load_gather(ad_t, [dv])
                e = av + bv
                e = jnp.maximum(e, 0.2 * e)
                ee = jnp.exp(e)
                ids = base + k * LANES + lax.iota(jnp.int32, LANES)
                ee = jnp.where(ids < EP, ee, 0.0)
                eebuf[pl.ds(k * LANES, LANES)] = ee
                srcbuf[pl.ds(k * LANES, LANES)] = sv + head * N

            pltpu.async_copy(xw2d.at[srcbuf], rows_st, sem).wait()

            def edge_body(ei, _):
                sb = plsc.load_gather(eebuf, [jnp.broadcast_to(ei, (LANES,))])
                for k in range(D // LANES):
                    rows_st[ei, pl.ds(k * LANES, LANES)] = (
                        rows_st[ei, pl.ds(k * LANES, LANES)] * sb)
                eerow[ei, pl.ds(0, LANES)] = sb
                return 0
            lax.fori_loop(0, CHUNK, edge_body, 0)

            pltpu.sync_copy(rows_st, acc_sh.at[dstbuf.at[0]], add=True)
            pltpu.sync_copy(eerow, den_sh.at[dstbuf.at[0]], add=True)
            return 0

        lax.fori_loop(0, NCHUNK, chunk_body, 0)

        plsc.subcore_barrier()

        pltpu.sync_copy(acc_sh.at[pl.ds(sid * ROWS_T, ROWS_T)],
                        feat.at[head, pl.ds(sid * ROWS_T, ROWS_T)])
        pltpu.sync_copy(den_sh.at[pl.ds(sid * ROWS_T, ROWS_T)],
                        den.at[head, pl.ds(sid * ROWS_T, ROWS_T)])


def _sc_edge(xw, asl, adl, src2d, dst2d):
    mesh = plsc.VectorSubcoreMesh(core_axis_name="c", subcore_axis_name="s")
    fn = pl.kernel(
        _sc_edge_body,
        out_type=[
            jax.ShapeDtypeStruct((H, N, D), jnp.float32),
            jax.ShapeDtypeStruct((H, N, LANES), jnp.float32),
        ],
        mesh=mesh,
        compiler_params=pltpu.CompilerParams(
            use_tc_tiling_on_sc=False, needs_layout_passes=False),
        scratch_types=[
            pltpu.VMEM_SHARED((N, D), jnp.float32),
            pltpu.VMEM_SHARED((N, LANES), jnp.float32),
            pltpu.VMEM((N,), jnp.float32),
            pltpu.VMEM((N,), jnp.float32),
            pltpu.VMEM((CHUNK,), jnp.int32),
            pltpu.VMEM((1, CHUNK), jnp.int32),
            pltpu.VMEM((CHUNK, D), jnp.float32),
            pltpu.VMEM((CHUNK, LANES), jnp.float32),
            pltpu.VMEM((CHUNK,), jnp.float32),
            pltpu.SemaphoreType.DMA,
        ],
    )
    return fn(xw.reshape(H * N, D), asl, adl, src2d, dst2d)


def kernel(x, edge_index, W, att_src, att_dst, bias, bn_gamma, bn_beta,
           ens_W, ens_b):
    loop = jnp.arange(N, dtype=edge_index.dtype)
    src = jnp.concatenate([edge_index[0], loop])
    dst = jnp.concatenate([edge_index[1], loop])
    pad = EP_PAD - EP
    src2d = jnp.pad(src, (0, pad)).reshape(RC, CHUNK)
    dst2d = jnp.pad(dst, (0, pad)).reshape(RC, CHUNK)

    inv = 1.0 / jnp.sqrt(1.0 + BN_EPS)
    scale = bn_gamma * inv
    shift = bias * scale + bn_beta
    ensW = ens_W.reshape(H, D, C)

    xw, asq, adq = _tc_pre1(x, W[:, 0], att_src[:, 0], att_dst[:, 0])
    feat1, den1 = _sc_edge(xw, asq.T, adq.T, src2d, dst2d)
    xw2, asq2, adq2 = _tc_pre2(feat1, den1, W[:, 1], att_src[:, 1],
                               att_dst[:, 1], scale[:, 0], shift[:, 0])
    feat2, den2 = _sc_edge(xw2, asq2.T, adq2.T, src2d, dst2d)
    return _tc_final(feat2, den2, scale[:, 1], shift[:, 1], ensW, ens_b)
